# Initial kernel scaffold; baseline (speedup 1.0000x reference)
import jax
import jax.numpy as jnp
from jax.experimental import pallas as pl


def kernel(edge_index, edge_type, query, relations, Ws, bs, query_weight):
    raise NotImplementedError("write your pallas kernel here")


# speedup vs baseline: 1.9071x; 1.9071x over previous
import functools

import jax
import jax.numpy as jnp
import numpy as np
from jax.experimental import pallas as pl

_N_NODES = 50000
_N_REL = 237
_DIM = 32
_N_LAYERS = 6

_BN = 2000
_INTERPRET = False


def _dense_body(h_ref, sums_ref, sqs_ref, mx_ref, mn_ref, aux_ref, hidx_ref,
                qrow_ref, a_ref, b_ref, bias_ref, out_ref):
    h = h_ref[...]
    rd = aux_ref[:, 0:1]
    s1 = aux_ref[:, 1:2]
    s2 = aux_ref[:, 2:3]
    i = pl.program_id(0)
    ids = i * _BN + jax.lax.broadcasted_iota(jnp.int32, (_BN, 1), 0)
    hm = jnp.any(ids == hidx_ref[...], axis=1, keepdims=True)
    bvec = jnp.where(hm, qrow_ref[...], 0.0)
    sums = sums_ref[...] + bvec
    sqs = sqs_ref[...] + bvec * bvec
    mx = jnp.maximum(mx_ref[...], bvec)
    mn = jnp.minimum(mn_ref[...], bvec)
    mean = sums * rd
    sqm = sqs * rd
    std = jnp.sqrt(jnp.clip(sqm - mean * mean, 1e-6, None))
    feat = jnp.concatenate([mean, mx, mn, std], axis=1)
    fs = jnp.concatenate([feat, feat * s1, feat * s2], axis=1)
    acc = (jnp.dot(h, a_ref[...], preferred_element_type=jnp.float32)
           + jnp.dot(fs, b_ref[...], preferred_element_type=jnp.float32)
           + bias_ref[...])
    out_ref[...] = jnp.maximum(acc, 0.0) + h


def _dense_update(h, sums, sqs, mx, mn, aux, hidx, qrow, A, B, bias):
    grid = (_N_NODES // _BN,)
    bspec_nodes = pl.BlockSpec((_BN, _DIM), lambda i: (i, 0))
    return pl.pallas_call(
        _dense_body,
        grid=grid,
        in_specs=[
            bspec_nodes,
            bspec_nodes,
            bspec_nodes,
            bspec_nodes,
            bspec_nodes,
            pl.BlockSpec((_BN, 8), lambda i: (i, 0)),
            pl.BlockSpec((1, 64), lambda i: (0, 0)),
            pl.BlockSpec((1, _DIM), lambda i: (0, 0)),
            pl.BlockSpec((_DIM, _DIM), lambda i: (0, 0)),
            pl.BlockSpec((12 * _DIM, _DIM), lambda i: (0, 0)),
            pl.BlockSpec((1, _DIM), lambda i: (0, 0)),
        ],
        out_specs=bspec_nodes,
        out_shape=jax.ShapeDtypeStruct((_N_NODES, _DIM), jnp.float32),
        interpret=_INTERPRET,
    )(h, sums, sqs, mx, mn, aux, hidx, qrow, A, B, bias)


def kernel(edge_index, edge_type, query, relations, Ws, bs, query_weight):
    h_index = query[:, 0]
    t_index = query[:, 1]
    src = jnp.concatenate([edge_index[0], edge_index[1]])
    dst = jnp.concatenate([edge_index[1], edge_index[0]])
    etype = jnp.concatenate([edge_type, edge_type + _N_REL])

    WT = jnp.swapaxes(Ws, 1, 2)
    A_all = WT[:, :_DIM, :]
    fprime = np.arange(4 * _DIM)
    fidx = 4 * (fprime % _DIM) + (fprime // _DIM)
    Bs = [WT[:, _DIM + 3 * fidx + s, :] for s in range(3)]
    B_all = jnp.concatenate(Bs, axis=1)
    bias_all = bs[:, None, :]

    deg = jax.ops.segment_sum(jnp.ones_like(dst, jnp.float32), dst,
                              num_segments=_N_NODES) + 1.0
    scale = jnp.log(deg)
    scale = scale / jnp.mean(scale)
    aux = jnp.zeros((_N_NODES, 8), jnp.float32)
    aux = aux.at[:, 0].set(1.0 / deg)
    aux = aux.at[:, 1].set(scale)
    aux = aux.at[:, 2].set(1.0 / jnp.clip(scale, 1e-2, None))

    hidx = h_index.reshape(1, 64)
    qrow = query_weight[0:1, :]

    h = jnp.zeros((_N_NODES, _DIM), jnp.float32)
    h = h.at[h_index].set(jnp.broadcast_to(qrow, (64, _DIM)))

    for l in range(_N_LAYERS):
        msg = h[src] * relations[l][etype]
        sums = jax.ops.segment_sum(msg, dst, num_segments=_N_NODES)
        sqs = jax.ops.segment_sum(msg * msg, dst, num_segments=_N_NODES)
        mx = jax.ops.segment_max(msg, dst, num_segments=_N_NODES)
        mn = jax.ops.segment_min(msg, dst, num_segments=_N_NODES)
        h = _dense_update(h, sums, sqs, mx, mn, aux, hidx, qrow,
                          A_all[l], B_all[l], bias_all[l])
    return h[t_index]


# speedup vs baseline: 2.2261x; 1.1672x over previous
import functools

import jax
import jax.numpy as jnp
import numpy as np
from jax.experimental import pallas as pl

_N_NODES = 50000
_N_REL = 237
_DIM = 32
_N_LAYERS = 6

_BN = 2000
_INTERPRET = False


def _dense_body(h_ref, sums_ref, sqs_ref, mx_ref, mn_ref, aux_ref, hidx_ref,
                qrow_ref, a_ref, b_ref, bias_ref, out_ref):
    h = h_ref[...]
    rd = aux_ref[:, 0:1]
    s1 = aux_ref[:, 1:2]
    s2 = aux_ref[:, 2:3]
    i = pl.program_id(0)
    ids = i * _BN + jax.lax.broadcasted_iota(jnp.int32, (_BN, 1), 0)
    hm = jnp.any(ids == hidx_ref[...], axis=1, keepdims=True)
    bvec = jnp.where(hm, qrow_ref[...], 0.0)
    sums = sums_ref[...] + bvec
    sqs = sqs_ref[...] + bvec * bvec
    mx = jnp.maximum(mx_ref[...], bvec)
    mn = jnp.minimum(mn_ref[...], bvec)
    mean = sums * rd
    sqm = sqs * rd
    std = jnp.sqrt(jnp.clip(sqm - mean * mean, 1e-6, None))
    feat = jnp.concatenate([mean, mx, mn, std], axis=1)
    fs = jnp.concatenate([feat, feat * s1, feat * s2], axis=1)
    acc = (jnp.dot(h, a_ref[...], preferred_element_type=jnp.float32)
           + jnp.dot(fs, b_ref[...], preferred_element_type=jnp.float32)
           + bias_ref[...])
    out_ref[...] = jnp.maximum(acc, 0.0) + h


def _dense_update(h, sums, sqs, mx, mn, aux, hidx, qrow, A, B, bias):
    grid = (_N_NODES // _BN,)
    bspec_nodes = pl.BlockSpec((_BN, _DIM), lambda i: (i, 0))
    return pl.pallas_call(
        _dense_body,
        grid=grid,
        in_specs=[
            bspec_nodes,
            bspec_nodes,
            bspec_nodes,
            bspec_nodes,
            bspec_nodes,
            pl.BlockSpec((_BN, 8), lambda i: (i, 0)),
            pl.BlockSpec((1, 64), lambda i: (0, 0)),
            pl.BlockSpec((1, _DIM), lambda i: (0, 0)),
            pl.BlockSpec((_DIM, _DIM), lambda i: (0, 0)),
            pl.BlockSpec((12 * _DIM, _DIM), lambda i: (0, 0)),
            pl.BlockSpec((1, _DIM), lambda i: (0, 0)),
        ],
        out_specs=bspec_nodes,
        out_shape=jax.ShapeDtypeStruct((_N_NODES, _DIM), jnp.float32),
        interpret=_INTERPRET,
    )(h, sums, sqs, mx, mn, aux, hidx, qrow, A, B, bias)


def kernel(edge_index, edge_type, query, relations, Ws, bs, query_weight):
    h_index = query[:, 0]
    t_index = query[:, 1]
    src = jnp.concatenate([edge_index[0], edge_index[1]])
    dst = jnp.concatenate([edge_index[1], edge_index[0]])
    etype = jnp.concatenate([edge_type, edge_type + _N_REL])
    order = jnp.argsort(dst)
    src = src[order]
    dst = dst[order]
    etype = etype[order]

    WT = jnp.swapaxes(Ws, 1, 2)
    A_all = WT[:, :_DIM, :]
    fprime = np.arange(4 * _DIM)
    fidx = 4 * (fprime % _DIM) + (fprime // _DIM)
    Bs = [WT[:, _DIM + 3 * fidx + s, :] for s in range(3)]
    B_all = jnp.concatenate(Bs, axis=1)
    bias_all = bs[:, None, :]

    deg = jax.ops.segment_sum(jnp.ones_like(dst, jnp.float32), dst,
                              num_segments=_N_NODES) + 1.0
    scale = jnp.log(deg)
    scale = scale / jnp.mean(scale)
    aux = jnp.zeros((_N_NODES, 8), jnp.float32)
    aux = aux.at[:, 0].set(1.0 / deg)
    aux = aux.at[:, 1].set(scale)
    aux = aux.at[:, 2].set(1.0 / jnp.clip(scale, 1e-2, None))

    hidx = h_index.reshape(1, 64)
    qrow = query_weight[0:1, :]

    h = jnp.zeros((_N_NODES, _DIM), jnp.float32)
    h = h.at[h_index].set(jnp.broadcast_to(qrow, (64, _DIM)))

    for l in range(_N_LAYERS):
        msg = h[src] * relations[l][etype]
        sums = jax.ops.segment_sum(msg, dst, num_segments=_N_NODES, indices_are_sorted=True)
        sqs = jax.ops.segment_sum(msg * msg, dst, num_segments=_N_NODES, indices_are_sorted=True)
        mx = jax.ops.segment_max(msg, dst, num_segments=_N_NODES, indices_are_sorted=True)
        mn = jax.ops.segment_min(msg, dst, num_segments=_N_NODES, indices_are_sorted=True)
        h = _dense_update(h, sums, sqs, mx, mn, aux, hidx, qrow,
                          A_all[l], B_all[l], bias_all[l])
    return h[t_index]


# speedup vs baseline: 5.0495x; 2.2683x over previous
import functools

import jax
import jax.numpy as jnp
import numpy as np
from jax import lax
from jax.experimental import pallas as pl
from jax.experimental.pallas import tpu as pltpu
from jax.experimental.pallas import tpu_sc as plsc

_N_NODES = 50000
_N_REL = 237
_NRELT = 2 * _N_REL
_DIM = 32
_N_LAYERS = 6
_N_EDGES = 1600000

_NW = 32
_N_PASS = 4
_NVW = _NW * _N_PASS
_NSUB = 392
_N_PAD = _NVW * _NSUB
_EB = 256
_E_GUARD = 2048
_E_PAD = _N_EDGES + _E_GUARD
_ACC_ROWS = 400
_BN = 1792
_INTERPRET = False

_NEG = float(np.finfo(np.float32).min)
_POS = float(np.finfo(np.float32).max)


def _sc_msg_body(h_hbm, src_hbm, et_hbm, dst_hbm, bounds_hbm, rel_hbm,
                 out_hbm, rel_v, bounds_v, sidx0, sidx1, eidx0, eidx1,
                 dstb0, dstb1, hrows0, hrows1, acc_v, sem0, sem1):
    cid = lax.axis_index("c")
    sid = lax.axis_index("s")
    wid = sid * 2 + cid
    pltpu.sync_copy(rel_hbm, rel_v)
    pltpu.sync_copy(bounds_hbm, bounds_v)
    sems = (sem0, sem1)
    sidxs = (sidx0, sidx1)
    eidxs = (eidx0, eidx1)
    dstbs = (dstb0, dstb1)
    hrowss = (hrows0, hrows1)

    zero = jnp.zeros((16,), jnp.float32)
    negv = jnp.full((16,), _NEG, jnp.float32)
    posv = jnp.full((16,), _POS, jnp.float32)

    def stage(eb, slot):
        eb = pl.multiple_of(eb, 8)
        pltpu.sync_copy(src_hbm.at[pl.ds(eb, _EB)], sidxs[slot])
        pltpu.sync_copy(et_hbm.at[pl.ds(eb, _EB)], eidxs[slot])
        pltpu.sync_copy(dst_hbm.at[pl.ds(eb, _EB)], dstbs[slot])
        pltpu.async_copy(h_hbm.at[sidxs[slot]], hrowss[slot], sems[slot])

    def wait(slot):
        pltpu.make_async_copy(h_hbm.at[sidxs[slot]], hrowss[slot],
                              sems[slot]).wait()

    def one_pass(p, _):
        w = p * _NW + wid
        n0 = w * _NSUB
        bv = bounds_v[pl.ds(w, 16)]
        e0 = bv[0]
        e1 = bv[1]
        e0a = jnp.bitwise_and(e0, -8)
        nb2 = jnp.maximum((e1 - e0a + 2 * _EB - 1) // (2 * _EB), 1)

        def init_row(r, carry):
            acc_v[r, pl.ds(0, 16)] = zero
            acc_v[r, pl.ds(16, 16)] = zero
            acc_v[r, pl.ds(32, 16)] = zero
            acc_v[r, pl.ds(48, 16)] = zero
            acc_v[r, pl.ds(64, 16)] = negv
            acc_v[r, pl.ds(80, 16)] = negv
            acc_v[r, pl.ds(96, 16)] = posv
            acc_v[r, pl.ds(112, 16)] = posv
            return carry
        lax.fori_loop(0, _ACC_ROWS - 7, init_row, 0)

        stage(e0a, 0)

        def compute(slot, carry):
            def edges16(ib, carry2):
                (dprev, s0, s1, q0, q1, x0, x1, m0, m1) = carry2
                dvec = dstbs[slot][pl.ds(ib * 16, 16)]
                evec = eidxs[slot][pl.ds(ib * 16, 16)]
                for j in range(16):
                    e = ib * 16 + j
                    d = dvec[j]
                    et = evec[j]
                    h0 = hrowss[slot][e, pl.ds(0, 16)]
                    h1 = hrowss[slot][e, pl.ds(16, 16)]
                    r0 = rel_v[et, pl.ds(0, 16)]
                    r1 = rel_v[et, pl.ds(16, 16)]
                    g0 = h0 * r0
                    g1 = h1 * r1
                    new = d != dprev
                    lrel = d - n0
                    inr = jnp.logical_and(lrel >= 0, lrel < _NSUB)
                    loc = jnp.where(inr, lrel, _NSUB)
                    s0 = jnp.where(new, g0, s0 + g0)
                    s1 = jnp.where(new, g1, s1 + g1)
                    q0 = jnp.where(new, g0 * g0, q0 + g0 * g0)
                    q1 = jnp.where(new, g1 * g1, q1 + g1 * g1)
                    x0 = jnp.where(new, g0, jnp.maximum(x0, g0))
                    x1 = jnp.where(new, g1, jnp.maximum(x1, g1))
                    m0 = jnp.where(new, g0, jnp.minimum(m0, g0))
                    m1 = jnp.where(new, g1, jnp.minimum(m1, g1))
                    acc_v[loc, pl.ds(0, 16)] = s0
                    acc_v[loc, pl.ds(16, 16)] = s1
                    acc_v[loc, pl.ds(32, 16)] = q0
                    acc_v[loc, pl.ds(48, 16)] = q1
                    acc_v[loc, pl.ds(64, 16)] = x0
                    acc_v[loc, pl.ds(80, 16)] = x1
                    acc_v[loc, pl.ds(96, 16)] = m0
                    acc_v[loc, pl.ds(112, 16)] = m1
                    dprev = d
                return (dprev, s0, s1, q0, q1, x0, x1, m0, m1)
            return lax.fori_loop(0, _EB // 16, edges16, carry)

        carry0 = (jnp.int32(-1), zero, zero, zero, zero, zero, zero, zero,
                  zero)

        def pair(g2, carry):
            nxt = 2 * g2 + 1

            @pl.when(nxt < 2 * nb2)
            def _():
                stage(e0a + nxt * _EB, 1)
            wait(0)
            carry = compute(0, carry)
            nxt2 = 2 * g2 + 2

            @pl.when(nxt2 < 2 * nb2)
            def _():
                stage(e0a + nxt2 * _EB, 0)
            wait(1)
            carry = compute(1, carry)
            return carry

        lax.fori_loop(0, nb2, pair, carry0)
        pltpu.sync_copy(acc_v.at[pl.ds(0, _NSUB)],
                        out_hbm.at[pl.ds(n0, _NSUB)])
        return 0

    lax.fori_loop(0, _N_PASS, one_pass, 0)


def _sc_msg(h, srcp, etp, dstp, bounds, rel):
    kfn = functools.partial(
        pl.kernel,
        mesh=plsc.VectorSubcoreMesh(core_axis_name="c", subcore_axis_name="s"),
        compiler_params=pltpu.CompilerParams(use_tc_tiling_on_sc=False),
        out_type=jax.ShapeDtypeStruct((_N_PAD, 128), jnp.float32),
        scratch_types=[
            pltpu.VMEM((_NRELT, _DIM), jnp.float32),
            pltpu.VMEM((144,), jnp.int32),
            pltpu.VMEM((_EB,), jnp.int32),
            pltpu.VMEM((_EB,), jnp.int32),
            pltpu.VMEM((_EB,), jnp.int32),
            pltpu.VMEM((_EB,), jnp.int32),
            pltpu.VMEM((_EB,), jnp.int32),
            pltpu.VMEM((_EB,), jnp.int32),
            pltpu.VMEM((_EB, _DIM), jnp.float32),
            pltpu.VMEM((_EB, _DIM), jnp.float32),
            pltpu.VMEM((_ACC_ROWS, 128), jnp.float32),
            pltpu.SemaphoreType.DMA,
            pltpu.SemaphoreType.DMA,
        ],
    )(_sc_msg_body)
    return kfn(h, srcp, etp, dstp, bounds, rel)


def _dense_body(h_ref, agg_ref, aux_ref, hidx_ref, qrow_ref, a_ref, b_ref,
                bias_ref, out_ref):
    h = h_ref[...]
    agg = agg_ref[...]
    rd = aux_ref[:, 0:1]
    s1 = aux_ref[:, 1:2]
    s2 = aux_ref[:, 2:3]
    i = pl.program_id(0)
    ids = i * _BN + jax.lax.broadcasted_iota(jnp.int32, (_BN, 1), 0)
    hm = jnp.any(ids == hidx_ref[...], axis=1, keepdims=True)
    bvec = jnp.where(hm, qrow_ref[...], 0.0)
    sums = agg[:, 0:32] + bvec
    sqs = agg[:, 32:64] + bvec * bvec
    mx = jnp.maximum(agg[:, 64:96], bvec)
    mn = jnp.minimum(agg[:, 96:128], bvec)
    mean = sums * rd
    sqm = sqs * rd
    std = jnp.sqrt(jnp.clip(sqm - mean * mean, 1e-6, None))
    feat = jnp.concatenate([mean, mx, mn, std], axis=1)
    fs = jnp.concatenate([feat, feat * s1, feat * s2], axis=1)
    acc = (jnp.dot(h, a_ref[...], preferred_element_type=jnp.float32)
           + jnp.dot(fs, b_ref[...], preferred_element_type=jnp.float32)
           + bias_ref[...])
    out_ref[...] = jnp.maximum(acc, 0.0) + h


def _dense_update(h, agg, aux, hidx, qrow, A, B, bias):
    grid = (_N_PAD // _BN,)
    bspec_nodes = pl.BlockSpec((_BN, _DIM), lambda i: (i, 0))
    return pl.pallas_call(
        _dense_body,
        grid=grid,
        in_specs=[
            bspec_nodes,
            pl.BlockSpec((_BN, 128), lambda i: (i, 0)),
            pl.BlockSpec((_BN, 8), lambda i: (i, 0)),
            pl.BlockSpec((1, 64), lambda i: (0, 0)),
            pl.BlockSpec((1, _DIM), lambda i: (0, 0)),
            pl.BlockSpec((_DIM, _DIM), lambda i: (0, 0)),
            pl.BlockSpec((12 * _DIM, _DIM), lambda i: (0, 0)),
            pl.BlockSpec((1, _DIM), lambda i: (0, 0)),
        ],
        out_specs=bspec_nodes,
        out_shape=jax.ShapeDtypeStruct((_N_PAD, _DIM), jnp.float32),
        interpret=_INTERPRET,
    )(h, agg, aux, hidx, qrow, A, B, bias)


def kernel(edge_index, edge_type, query, relations, Ws, bs, query_weight):
    h_index = query[:, 0]
    t_index = query[:, 1]
    src = jnp.concatenate([edge_index[0], edge_index[1]])
    dst = jnp.concatenate([edge_index[1], edge_index[0]])
    etype = jnp.concatenate([edge_type, edge_type + _N_REL])

    order = jnp.argsort(dst)
    dsts = dst[order]
    srcs = src[order]
    ets = etype[order]
    rowptr = jnp.searchsorted(dsts, jnp.arange(_N_PAD + 1,
                                               dtype=jnp.int32)).astype(jnp.int32)
    bounds = jnp.zeros((144,), jnp.int32)
    bounds = bounds.at[: _NVW + 1].set(rowptr[:: _NSUB])
    srcp = jnp.concatenate([srcs, jnp.zeros((_E_GUARD,), jnp.int32)])
    etp = jnp.concatenate([ets, jnp.zeros((_E_GUARD,), jnp.int32)])
    dstp = jnp.concatenate(
        [dsts, jnp.full((_E_GUARD,), _N_PAD - 1, jnp.int32)])

    WT = jnp.swapaxes(Ws, 1, 2)
    A_all = WT[:, :_DIM, :]
    fprime = np.arange(4 * _DIM)
    fidx = 4 * (fprime % _DIM) + (fprime // _DIM)
    Bs = [WT[:, _DIM + 3 * fidx + s, :] for s in range(3)]
    B_all = jnp.concatenate(Bs, axis=1)
    bias_all = bs[:, None, :]

    deg = (rowptr[1:_N_NODES + 1] - rowptr[:_N_NODES]).astype(jnp.float32) + 1.0
    scale = jnp.log(deg)
    scale = scale / jnp.mean(scale)
    scale = jnp.concatenate(
        [scale, jnp.ones((_N_PAD - _N_NODES,), jnp.float32)])
    degp = jnp.concatenate([deg, jnp.ones((_N_PAD - _N_NODES,), jnp.float32)])
    aux = jnp.zeros((_N_PAD, 8), jnp.float32)
    aux = aux.at[:, 0].set(1.0 / degp)
    aux = aux.at[:, 1].set(scale)
    aux = aux.at[:, 2].set(1.0 / jnp.clip(scale, 1e-2, None))

    hidx = h_index.reshape(1, 64)
    qrow = query_weight[0:1, :]

    h = jnp.zeros((_N_PAD, _DIM), jnp.float32)
    h = h.at[h_index].set(jnp.broadcast_to(qrow, (64, _DIM)))

    for l in range(_N_LAYERS):
        agg = _sc_msg(h, srcp, etp, dstp, bounds, relations[l])
        h = _dense_update(h, agg, aux, hidx, qrow, A_all[l], B_all[l],
                          bias_all[l])
    return h[t_index]


# speedup vs baseline: 23.8545x; 4.7241x over previous
import functools

import jax
import jax.numpy as jnp
import numpy as np
from jax import lax
from jax.experimental import pallas as pl
from jax.experimental.pallas import tpu as pltpu
from jax.experimental.pallas import tpu_sc as plsc

_N_NODES = 50000
_N_REL = 237
_NRELT = 2 * _N_REL
_DIM = 32
_N_LAYERS = 6
_N_EDGES = 1600000

_NW = 32
_N_PASS = 4
_NVW = _NW * _N_PASS
_NSUB = 392
_N_PAD = _NVW * _NSUB
_EB = 256
_E_GUARD = 2048
_E_PAD = _N_EDGES + _E_GUARD
_ACC_ROWS = 400
_BN = 1792
_INTERPRET = False

_NEG = float(np.finfo(np.float32).min)
_POS = float(np.finfo(np.float32).max)


def _sc_msg_body(h_hbm, src_hbm, et_hbm, dst_hbm, bounds_hbm, rel_hbm,
                 out_hbm, rel_v, bounds_v, sidx0, sidx1, eidx0, eidx1,
                 dstb0, dstb1, hrows0, hrows1, acc_v, sem0, sem1):
    cid = lax.axis_index("c")
    sid = lax.axis_index("s")
    wid = sid * 2 + cid
    pltpu.sync_copy(rel_hbm, rel_v)
    pltpu.sync_copy(bounds_hbm, bounds_v)
    sems = (sem0, sem1)
    sidxs = (sidx0, sidx1)
    eidxs = (eidx0, eidx1)
    dstbs = (dstb0, dstb1)
    hrowss = (hrows0, hrows1)

    zero = jnp.zeros((16,), jnp.float32)
    negv = jnp.full((16,), _NEG, jnp.float32)
    posv = jnp.full((16,), _POS, jnp.float32)

    def stage(eb, slot):
        eb = pl.multiple_of(eb, 8)
        pltpu.sync_copy(src_hbm.at[pl.ds(eb, _EB)], sidxs[slot])
        pltpu.sync_copy(et_hbm.at[pl.ds(eb, _EB)], eidxs[slot])
        pltpu.sync_copy(dst_hbm.at[pl.ds(eb, _EB)], dstbs[slot])
        pltpu.async_copy(h_hbm.at[sidxs[slot]], hrowss[slot], sems[slot])

    def wait(slot):
        pltpu.make_async_copy(h_hbm.at[sidxs[slot]], hrowss[slot],
                              sems[slot]).wait()

    def one_pass(p, _):
        w = p * _NW + wid
        n0 = w * _NSUB
        bv = bounds_v[pl.ds(w, 16)]
        e0 = bv[0]
        e1 = bv[1]
        e0a = jnp.bitwise_and(e0, -8)
        nb2 = jnp.maximum((e1 - e0a + 2 * _EB - 1) // (2 * _EB), 1)

        def init_row(r, carry):
            acc_v[r, pl.ds(0, 16)] = zero
            acc_v[r, pl.ds(16, 16)] = zero
            acc_v[r, pl.ds(32, 16)] = zero
            acc_v[r, pl.ds(48, 16)] = zero
            acc_v[r, pl.ds(64, 16)] = negv
            acc_v[r, pl.ds(80, 16)] = negv
            acc_v[r, pl.ds(96, 16)] = posv
            acc_v[r, pl.ds(112, 16)] = posv
            return carry
        lax.fori_loop(0, _ACC_ROWS - 7, init_row, 0)

        stage(e0a, 0)

        def compute(slot, carry):
            def edges16(ib, carry2):
                (dprev, s0, s1, q0, q1, x0, x1, m0, m1) = carry2
                dvec = dstbs[slot][pl.ds(ib * 16, 16)]
                evec = eidxs[slot][pl.ds(ib * 16, 16)]
                for j in range(16):
                    e = ib * 16 + j
                    d = dvec[j]
                    et = evec[j]
                    h0 = hrowss[slot][e, pl.ds(0, 16)]
                    h1 = hrowss[slot][e, pl.ds(16, 16)]
                    r0 = rel_v[et, pl.ds(0, 16)]
                    r1 = rel_v[et, pl.ds(16, 16)]
                    g0 = h0 * r0
                    g1 = h1 * r1
                    new = d != dprev
                    lrel = d - n0
                    inr = jnp.logical_and(lrel >= 0, lrel < _NSUB)
                    loc = jnp.where(inr, lrel, _NSUB)
                    s0 = jnp.where(new, g0, s0 + g0)
                    s1 = jnp.where(new, g1, s1 + g1)
                    q0 = jnp.where(new, g0 * g0, q0 + g0 * g0)
                    q1 = jnp.where(new, g1 * g1, q1 + g1 * g1)
                    x0 = jnp.where(new, g0, jnp.maximum(x0, g0))
                    x1 = jnp.where(new, g1, jnp.maximum(x1, g1))
                    m0 = jnp.where(new, g0, jnp.minimum(m0, g0))
                    m1 = jnp.where(new, g1, jnp.minimum(m1, g1))
                    acc_v[loc, pl.ds(0, 16)] = s0
                    acc_v[loc, pl.ds(16, 16)] = s1
                    acc_v[loc, pl.ds(32, 16)] = q0
                    acc_v[loc, pl.ds(48, 16)] = q1
                    acc_v[loc, pl.ds(64, 16)] = x0
                    acc_v[loc, pl.ds(80, 16)] = x1
                    acc_v[loc, pl.ds(96, 16)] = m0
                    acc_v[loc, pl.ds(112, 16)] = m1
                    dprev = d
                return (dprev, s0, s1, q0, q1, x0, x1, m0, m1)
            return lax.fori_loop(0, _EB // 16, edges16, carry)

        carry0 = (jnp.int32(-1), zero, zero, zero, zero, zero, zero, zero,
                  zero)

        def pair(g2, carry):
            nxt = 2 * g2 + 1

            @pl.when(nxt < 2 * nb2)
            def _():
                stage(e0a + nxt * _EB, 1)
            wait(0)
            carry = compute(0, carry)
            nxt2 = 2 * g2 + 2

            @pl.when(nxt2 < 2 * nb2)
            def _():
                stage(e0a + nxt2 * _EB, 0)
            wait(1)
            carry = compute(1, carry)
            return carry

        lax.fori_loop(0, nb2, pair, carry0)
        pltpu.sync_copy(acc_v.at[pl.ds(0, _NSUB)],
                        out_hbm.at[pl.ds(n0, _NSUB)])
        return 0

    lax.fori_loop(0, _N_PASS, one_pass, 0)


def _sc_msg(h, srcp, etp, dstp, bounds, rel):
    kfn = functools.partial(
        pl.kernel,
        mesh=plsc.VectorSubcoreMesh(core_axis_name="c", subcore_axis_name="s"),
        compiler_params=pltpu.CompilerParams(use_tc_tiling_on_sc=False),
        out_type=jax.ShapeDtypeStruct((_N_PAD, 128), jnp.float32),
        scratch_types=[
            pltpu.VMEM((_NRELT, _DIM), jnp.float32),
            pltpu.VMEM((144,), jnp.int32),
            pltpu.VMEM((_EB,), jnp.int32),
            pltpu.VMEM((_EB,), jnp.int32),
            pltpu.VMEM((_EB,), jnp.int32),
            pltpu.VMEM((_EB,), jnp.int32),
            pltpu.VMEM((_EB,), jnp.int32),
            pltpu.VMEM((_EB,), jnp.int32),
            pltpu.VMEM((_EB, _DIM), jnp.float32),
            pltpu.VMEM((_EB, _DIM), jnp.float32),
            pltpu.VMEM((_ACC_ROWS, 128), jnp.float32),
            pltpu.SemaphoreType.DMA,
            pltpu.SemaphoreType.DMA,
        ],
    )(_sc_msg_body)
    return kfn(h, srcp, etp, dstp, bounds, rel)


def _dense_body(h_ref, agg_ref, aux_ref, hidx_ref, qrow_ref, a_ref, b_ref,
                bias_ref, out_ref):
    h = h_ref[...]
    agg = agg_ref[...]
    rd = aux_ref[:, 0:1]
    s1 = aux_ref[:, 1:2]
    s2 = aux_ref[:, 2:3]
    i = pl.program_id(0)
    ids = i * _BN + jax.lax.broadcasted_iota(jnp.int32, (_BN, 1), 0)
    hm = jnp.any(ids == hidx_ref[...], axis=1, keepdims=True)
    bvec = jnp.where(hm, qrow_ref[...], 0.0)
    sums = agg[:, 0:32] + bvec
    sqs = agg[:, 32:64] + bvec * bvec
    mx = jnp.maximum(agg[:, 64:96], bvec)
    mn = jnp.minimum(agg[:, 96:128], bvec)
    mean = sums * rd
    sqm = sqs * rd
    std = jnp.sqrt(jnp.clip(sqm - mean * mean, 1e-6, None))
    feat = jnp.concatenate([mean, mx, mn, std], axis=1)
    fs = jnp.concatenate([feat, feat * s1, feat * s2], axis=1)
    acc = (jnp.dot(h, a_ref[...], preferred_element_type=jnp.float32)
           + jnp.dot(fs, b_ref[...], preferred_element_type=jnp.float32)
           + bias_ref[...])
    out_ref[...] = jnp.maximum(acc, 0.0) + h


def _dense_update(h, agg, aux, hidx, qrow, A, B, bias):
    grid = (_N_PAD // _BN,)
    bspec_nodes = pl.BlockSpec((_BN, _DIM), lambda i: (i, 0))
    return pl.pallas_call(
        _dense_body,
        grid=grid,
        in_specs=[
            bspec_nodes,
            pl.BlockSpec((_BN, 128), lambda i: (i, 0)),
            pl.BlockSpec((_BN, 8), lambda i: (i, 0)),
            pl.BlockSpec((1, 64), lambda i: (0, 0)),
            pl.BlockSpec((1, _DIM), lambda i: (0, 0)),
            pl.BlockSpec((_DIM, _DIM), lambda i: (0, 0)),
            pl.BlockSpec((12 * _DIM, _DIM), lambda i: (0, 0)),
            pl.BlockSpec((1, _DIM), lambda i: (0, 0)),
        ],
        out_specs=bspec_nodes,
        out_shape=jax.ShapeDtypeStruct((_N_PAD, _DIM), jnp.float32),
        interpret=_INTERPRET,
    )(h, agg, aux, hidx, qrow, A, B, bias)


def kernel(edge_index, edge_type, query, relations, Ws, bs, query_weight):
    h_index = query[:, 0]
    t_index = query[:, 1]
    src = jnp.concatenate([edge_index[0], edge_index[1]])
    dst = jnp.concatenate([edge_index[1], edge_index[0]])
    etype = jnp.concatenate([edge_type, edge_type + _N_REL])

    order = jnp.argsort(dst)
    dsts = dst[order]
    srcs = src[order]
    ets = etype[order]
    counts = jax.ops.segment_sum(jnp.ones_like(dsts), dsts,
                                 num_segments=_N_PAD, indices_are_sorted=True)
    rowptr = jnp.concatenate([jnp.zeros((1,), jnp.int32),
                              jnp.cumsum(counts).astype(jnp.int32)])
    bounds = jnp.zeros((144,), jnp.int32)
    bounds = bounds.at[: _NVW + 1].set(rowptr[:: _NSUB])
    srcp = jnp.concatenate([srcs, jnp.zeros((_E_GUARD,), jnp.int32)])
    etp = jnp.concatenate([ets, jnp.zeros((_E_GUARD,), jnp.int32)])
    dstp = jnp.concatenate(
        [dsts, jnp.full((_E_GUARD,), _N_PAD - 1, jnp.int32)])

    WT = jnp.swapaxes(Ws, 1, 2)
    A_all = WT[:, :_DIM, :]
    fprime = np.arange(4 * _DIM)
    fidx = 4 * (fprime % _DIM) + (fprime // _DIM)
    Bs = [WT[:, _DIM + 3 * fidx + s, :] for s in range(3)]
    B_all = jnp.concatenate(Bs, axis=1)
    bias_all = bs[:, None, :]

    deg = (rowptr[1:_N_NODES + 1] - rowptr[:_N_NODES]).astype(jnp.float32) + 1.0
    scale = jnp.log(deg)
    scale = scale / jnp.mean(scale)
    scale = jnp.concatenate(
        [scale, jnp.ones((_N_PAD - _N_NODES,), jnp.float32)])
    degp = jnp.concatenate([deg, jnp.ones((_N_PAD - _N_NODES,), jnp.float32)])
    aux = jnp.zeros((_N_PAD, 8), jnp.float32)
    aux = aux.at[:, 0].set(1.0 / degp)
    aux = aux.at[:, 1].set(scale)
    aux = aux.at[:, 2].set(1.0 / jnp.clip(scale, 1e-2, None))

    hidx = h_index.reshape(1, 64)
    qrow = query_weight[0:1, :]

    h = jnp.zeros((_N_PAD, _DIM), jnp.float32)
    h = h.at[h_index].set(jnp.broadcast_to(qrow, (64, _DIM)))

    for l in range(_N_LAYERS):
        agg = _sc_msg(h, srcp, etp, dstp, bounds, relations[l])
        h = _dense_update(h, agg, aux, hidx, qrow, A_all[l], B_all[l],
                          bias_all[l])
    return h[t_index]
